# speedup vs baseline: 25.1310x; 25.1310x over previous
import jax
import jax.numpy as jnp
from jax.experimental import pallas as pl

ROWS = 16384
COLS = 200
DIM = 128
BLOCK_ROWS = 256


def _mask_embed_kernel(mask_ref, emb_ref, out_ref):
    m = mask_ref[...]
    e0 = emb_ref[0, :]
    e1 = emb_ref[1, :]
    out_ref[...] = jnp.where((m[:, :, None] != 0), e1[None, None, :],
                             e0[None, None, :])


def kernel(mask01, emb):
    grid = (ROWS // BLOCK_ROWS,)
    return pl.pallas_call(
        _mask_embed_kernel,
        grid=grid,
        in_specs=[
            pl.BlockSpec((BLOCK_ROWS, COLS), lambda i: (i, 0)),
            pl.BlockSpec((2, DIM), lambda i: (0, 0)),
        ],
        out_specs=pl.BlockSpec((BLOCK_ROWS, COLS, DIM), lambda i: (i, 0, 0)),
        out_shape=jax.ShapeDtypeStruct((ROWS, COLS, DIM), jnp.float32),
    )(mask01, emb)
